# Initial kernel scaffold; baseline (speedup 1.0000x reference)
import functools

import jax
import jax.numpy as jnp
from jax import lax
from jax.experimental import pallas as pl
from jax.experimental.pallas import tpu as pltpu
from jax.experimental.pallas import tpu_sc as plsc

N_FIELDS = 26
VOCAB = 100000
EMBED = 32
BATCH = 16384

NC, NS, L = 2, 16, 16
NW = NC * NS
TOTAL = BATCH * N_FIELDS
PER_W = TOTAL // NW
CHUNK = 1024
N_CH = PER_W // CHUNK

_mesh = plsc.VectorSubcoreMesh(
    core_axis_name="c", subcore_axis_name="s", num_cores=NC, num_subcores=NS
)


@functools.partial(
    pl.kernel,
    out_type=jax.ShapeDtypeStruct((TOTAL, EMBED), jnp.float32),
    mesh=_mesh,
    scratch_types=[
        pltpu.VMEM((PER_W,), jnp.int32),
        pltpu.VMEM((CHUNK, EMBED), jnp.float32),
        pltpu.SemaphoreType.DMA,
    ],
)
def _sc_gather(tables_hbm, tokens_hbm, out_hbm, idx_v, rows_v, g_sem):
    wid = lax.axis_index("s") * NC + lax.axis_index("c")
    base = wid * PER_W

    pltpu.sync_copy(tokens_hbm.at[pl.ds(base, PER_W)], idx_v)

    def body(j, carry):
        p0 = j * L
        lane = p0 + lax.iota(jnp.int32, L)
        field = lax.rem(lane, N_FIELDS)
        idx_v[pl.ds(p0, L)] = idx_v[pl.ds(p0, L)] + field * VOCAB
        return carry

    lax.fori_loop(0, PER_W // L, body, 0)

    for c in range(N_CH):
        pltpu.async_copy(
            tables_hbm.at[idx_v.at[pl.ds(c * CHUNK, CHUNK)]],
            rows_v,
            g_sem,
        ).wait()
        pltpu.sync_copy(rows_v, out_hbm.at[pl.ds(base + c * CHUNK, CHUNK)])


def kernel(tokens, tables):
    f = tables.shape[0]
    d = tables.shape[-1]
    tok_flat = tokens.reshape(-1).astype(jnp.int32)
    tab_flat = tables.reshape(-1, d)
    out = _sc_gather(tab_flat, tok_flat)
    return out.reshape(tokens.shape[0], f, d)


# speedup vs baseline: 1.1443x; 1.1443x over previous
import functools

import jax
import jax.numpy as jnp
from jax import lax
from jax.experimental import pallas as pl
from jax.experimental.pallas import tpu as pltpu
from jax.experimental.pallas import tpu_sc as plsc

N_FIELDS = 26
VOCAB = 100000
EMBED = 32
BATCH = 16384

NC, NS, L = 2, 16, 16
NW = NC * NS
TOTAL = BATCH * N_FIELDS
PER_W = TOTAL // NW
CHUNK = 1024
N_CH = PER_W // CHUNK

_mesh = plsc.VectorSubcoreMesh(
    core_axis_name="c", subcore_axis_name="s", num_cores=NC, num_subcores=NS
)


@functools.partial(
    pl.kernel,
    out_type=jax.ShapeDtypeStruct((TOTAL, EMBED), jnp.float32),
    mesh=_mesh,
    compiler_params=pltpu.CompilerParams(use_tc_tiling_on_sc=False),
    scratch_types=[
        pltpu.VMEM((PER_W,), jnp.int32),
        pltpu.VMEM((CHUNK, EMBED), jnp.float32),
        pltpu.SemaphoreType.DMA,
    ],
)
def _sc_gather(tables_hbm, tokens_hbm, out_hbm, idx_v, rows_v, g_sem):
    wid = lax.axis_index("s") * NC + lax.axis_index("c")
    base = wid * PER_W

    pltpu.sync_copy(tokens_hbm.at[pl.ds(base, PER_W)], idx_v)

    def body(j, carry):
        p0 = j * L
        lane = p0 + lax.iota(jnp.int32, L)
        field = lax.rem(lane, N_FIELDS)
        idx_v[pl.ds(p0, L)] = idx_v[pl.ds(p0, L)] + field * VOCAB
        return carry

    lax.fori_loop(0, PER_W // L, body, 0)

    for c in range(N_CH):
        pltpu.async_copy(
            tables_hbm.at[idx_v.at[pl.ds(c * CHUNK, CHUNK)]],
            rows_v,
            g_sem,
        ).wait()
        pltpu.sync_copy(rows_v, out_hbm.at[pl.ds(base + c * CHUNK, CHUNK)])


def kernel(tokens, tables):
    f = tables.shape[0]
    d = tables.shape[-1]
    tok_flat = tokens.reshape(-1).astype(jnp.int32)
    tab_flat = tables.reshape(-1, d)
    out = _sc_gather(tab_flat, tok_flat)
    return out.reshape(tokens.shape[0], f, d)


# speedup vs baseline: 1.1512x; 1.0060x over previous
import functools

import jax
import jax.numpy as jnp
from jax import lax
from jax.experimental import pallas as pl
from jax.experimental.pallas import tpu as pltpu
from jax.experimental.pallas import tpu_sc as plsc

N_FIELDS = 26
VOCAB = 100000
EMBED = 32
BATCH = 16384

NC, NS, L = 2, 16, 16
NW = NC * NS
TOTAL = BATCH * N_FIELDS
PER_W = TOTAL // NW
CHUNK = 1024
N_CH = PER_W // CHUNK
NBUF = 3

_mesh = plsc.VectorSubcoreMesh(
    core_axis_name="c", subcore_axis_name="s", num_cores=NC, num_subcores=NS
)


@functools.partial(
    pl.kernel,
    out_type=jax.ShapeDtypeStruct((TOTAL, EMBED), jnp.float32),
    mesh=_mesh,
    compiler_params=pltpu.CompilerParams(use_tc_tiling_on_sc=False),
    scratch_types=[
        pltpu.VMEM((PER_W,), jnp.int32),
        pltpu.VMEM((NBUF, CHUNK, EMBED), jnp.float32),
        [pltpu.SemaphoreType.DMA] * NBUF,
        [pltpu.SemaphoreType.DMA] * NBUF,
    ],
)
def _sc_gather(tables_hbm, tokens_hbm, out_hbm, idx_v, rows_v, g_sems, o_sems):
    wid = lax.axis_index("s") * NC + lax.axis_index("c")
    base = wid * PER_W

    pltpu.sync_copy(tokens_hbm.at[pl.ds(base, PER_W)], idx_v)

    def body(j, carry):
        p0 = j * L
        lane = p0 + lax.iota(jnp.int32, L)
        field = lax.rem(lane, N_FIELDS)
        idx_v[pl.ds(p0, L)] = idx_v[pl.ds(p0, L)] + field * VOCAB
        return carry

    lax.fori_loop(0, PER_W // L, body, 0)

    def gather(c):
        return pltpu.async_copy(
            tables_hbm.at[idx_v.at[pl.ds(c * CHUNK, CHUNK)]],
            rows_v.at[c % NBUF],
            g_sems[c % NBUF],
        )

    def write_out(c):
        return pltpu.async_copy(
            rows_v.at[c % NBUF],
            out_hbm.at[pl.ds(base + c * CHUNK, CHUNK)],
            o_sems[c % NBUF],
        )

    g_h = [None] * N_CH
    o_h = [None] * N_CH
    for c in range(min(NBUF - 1, N_CH)):
        g_h[c] = gather(c)
    for c in range(N_CH):
        if c >= 1:
            o_h[c - 1].wait()
        nxt = c + NBUF - 1
        if nxt < N_CH:
            g_h[nxt] = gather(nxt)
        g_h[c].wait()
        o_h[c] = write_out(c)
    o_h[N_CH - 1].wait()


def kernel(tokens, tables):
    f = tables.shape[0]
    d = tables.shape[-1]
    tok_flat = tokens.reshape(-1).astype(jnp.int32)
    tab_flat = tables.reshape(-1, d)
    out = _sc_gather(tab_flat, tok_flat)
    return out.reshape(tokens.shape[0], f, d)
